# baseline (speedup 1.0000x reference)
import functools

import jax
import jax.numpy as jnp
from jax import lax
from jax.experimental import pallas as pl
from jax.experimental.pallas import tpu as pltpu
from jax.experimental.pallas import tpu_sc as plsc

B, T, D = 1024, 200, 64
N = B * T
NC, NS, L = 2, 16, 16
NW = NC * NS
BB = B // 128
UNITS = T * BB
CH = 128
PER_W = UNITS // NW
NBUF = 6
LOOK = 3

_mesh = plsc.VectorSubcoreMesh(core_axis_name="c", subcore_axis_name="s")


@functools.partial(
    pl.kernel,
    mesh=_mesh,
    out_type=jax.ShapeDtypeStruct((N, D), jnp.bfloat16),
    scratch_types=[
        pltpu.VMEM((2 * CH * PER_W,), jnp.int32),
        pltpu.VMEM((NBUF, CH, D), jnp.bfloat16),
        [pltpu.SemaphoreType.DMA] * NBUF,
        [pltpu.SemaphoreType.DMA] * NBUF,
        [pltpu.SemaphoreType.DMA] * NBUF,
    ],
    compiler_params=pltpu.CompilerParams(use_tc_tiling_on_sc=False,
                                         needs_layout_passes=False),
)
def _sc_embed(x_hbm, tab_hbm, out_hbm, xv, buf, ga, gb, gc):
    wid = lax.axis_index("s") * NC + lax.axis_index("c")
    ubase = wid * PER_W
    pltpu.sync_copy(x_hbm.at[pl.ds(ubase * 2 * CH, PER_W * 2 * CH)], xv)

    def gather(k, sem, off, add):
        b = k % NBUF
        pltpu.async_copy(tab_hbm.at[xv.at[pl.ds(2 * CH * k + off, CH)]],
                         buf.at[b], sem[b], add=add)

    def wait_gather(k, sem, off):
        b = k % NBUF
        pltpu.make_async_copy(tab_hbm.at[xv.at[pl.ds(2 * CH * k + off, CH)]],
                              buf.at[b], sem[b]).wait()

    def scatter(k):
        b = k % NBUF
        pltpu.async_copy(buf.at[b], out_hbm.at[pl.ds((ubase + k) * CH, CH)],
                         gc[b])

    def wait_scatter(k):
        b = k % NBUF
        pltpu.make_async_copy(buf.at[b], out_hbm.at[pl.ds((ubase + k) * CH, CH)],
                              gc[b]).wait()

    for k in range(LOOK):
        gather(k, ga, 0, False)
    for k in range(PER_W):
        if k + LOOK < PER_W:
            if k >= NBUF - LOOK:
                wait_scatter(k - (NBUF - LOOK))
            gather(k + LOOK, ga, 0, False)
        wait_gather(k, ga, 0)
        gather(k, gb, CH, True)
        wait_gather(k, gb, CH)
        scatter(k)
    for k in range(PER_W - NBUF, PER_W):
        wait_scatter(k)


def kernel(X, w_embed):
    xnat = (X.astype(jnp.int32)
             .reshape(BB, 128, T, 2)
             .transpose(2, 0, 3, 1)
             .reshape(2 * N))
    out = _sc_embed(xnat, w_embed.astype(jnp.bfloat16))
    h = out.reshape(T, B, D).transpose(1, 0, 2).astype(jnp.float32)
    return h, w_embed


# speedup vs baseline: 1.5127x; 1.5127x over previous
import functools

import jax
import jax.numpy as jnp
from jax import lax
from jax.experimental import pallas as pl
from jax.experimental.pallas import tpu as pltpu
from jax.experimental.pallas import tpu_sc as plsc

B, T, D = 1024, 200, 64
N = B * T
NC, NS, L = 2, 16, 16
NW = NC * NS
BB = B // 128
UNITS = T * BB
CH = 128
PER_W = UNITS // NW
NBUF = 6
LOOK = 3

_mesh = plsc.VectorSubcoreMesh(core_axis_name="c", subcore_axis_name="s")


@functools.partial(
    pl.kernel,
    mesh=_mesh,
    out_type=jax.ShapeDtypeStruct((N, D), jnp.float32),
    scratch_types=[
        pltpu.VMEM((2 * CH * PER_W,), jnp.int32),
        pltpu.VMEM((NBUF, CH, D), jnp.float32),
        [pltpu.SemaphoreType.DMA] * NBUF,
        [pltpu.SemaphoreType.DMA] * NBUF,
        [pltpu.SemaphoreType.DMA] * NBUF,
    ],
    compiler_params=pltpu.CompilerParams(use_tc_tiling_on_sc=False),
)
def _sc_embed(x_hbm, tab_hbm, out_hbm, xv, buf, ga, gb, gc):
    wid = lax.axis_index("s") * NC + lax.axis_index("c")
    ubase = wid * PER_W
    pltpu.sync_copy(x_hbm.at[pl.ds(ubase * 2 * CH, PER_W * 2 * CH)], xv)

    def gather(k, sem, off, add):
        b = k % NBUF
        pltpu.async_copy(tab_hbm.at[xv.at[pl.ds(2 * CH * k + off, CH)]],
                         buf.at[b], sem[b], add=add)

    def wait_gather(k, sem, off):
        b = k % NBUF
        pltpu.make_async_copy(tab_hbm.at[xv.at[pl.ds(2 * CH * k + off, CH)]],
                              buf.at[b], sem[b]).wait()

    def scatter(k):
        b = k % NBUF
        pltpu.async_copy(buf.at[b], out_hbm.at[pl.ds((ubase + k) * CH, CH)],
                         gc[b])

    def wait_scatter(k):
        b = k % NBUF
        pltpu.make_async_copy(buf.at[b], out_hbm.at[pl.ds((ubase + k) * CH, CH)],
                              gc[b]).wait()

    for k in range(LOOK):
        gather(k, ga, 0, False)
    for k in range(PER_W):
        if k + LOOK < PER_W:
            if k >= NBUF - LOOK:
                wait_scatter(k - (NBUF - LOOK))
            gather(k + LOOK, ga, 0, False)
        wait_gather(k, ga, 0)
        gather(k, gb, CH, True)
        wait_gather(k, gb, CH)
        scatter(k)
    for k in range(PER_W - NBUF, PER_W):
        wait_scatter(k)


def kernel(X, w_embed):
    xnat = (X.astype(jnp.int32)
             .reshape(BB, 128, T, 2)
             .transpose(2, 0, 3, 1)
             .reshape(2 * N))
    out = _sc_embed(xnat, w_embed)
    h = out.reshape(T, B, D).transpose(1, 0, 2)
    return h, w_embed


# speedup vs baseline: 1.5409x; 1.0187x over previous
import functools

import jax
import jax.numpy as jnp
from jax import lax
from jax.experimental import pallas as pl
from jax.experimental.pallas import tpu as pltpu
from jax.experimental.pallas import tpu_sc as plsc

B, T, D = 1024, 200, 64
N = B * T
NC, NS, L = 2, 16, 16
NW = NC * NS
PER_W = N // NW
CH = 400
G = PER_W // CH
NBUF = 4
LOOK = 2

_mesh = plsc.VectorSubcoreMesh(core_axis_name="c", subcore_axis_name="s")


@functools.partial(
    pl.kernel,
    mesh=_mesh,
    out_type=jax.ShapeDtypeStruct((N, D), jnp.float32),
    scratch_types=[
        pltpu.VMEM((1, G, CH), jnp.int32),
        pltpu.VMEM((1, G, CH), jnp.int32),
        pltpu.VMEM((NBUF, CH, D), jnp.float32),
        [pltpu.SemaphoreType.DMA] * NBUF,
        [pltpu.SemaphoreType.DMA] * NBUF,
        [pltpu.SemaphoreType.DMA] * NBUF,
    ],
    compiler_params=pltpu.CompilerParams(use_tc_tiling_on_sc=False),
)
def _sc_embed(idx0_hbm, idx1_hbm, tab_hbm, out_hbm,
              idx0_v, idx1_v, buf, ga, gb, gc):
    wid = lax.axis_index("s") * NC + lax.axis_index("c")
    gbase = wid * G
    pltpu.sync_copy(idx0_hbm.at[pl.ds(wid, 1)], idx0_v)
    pltpu.sync_copy(idx1_hbm.at[pl.ds(wid, 1)], idx1_v)

    def gather(j, idx_v, sem, add):
        b = j % NBUF
        pltpu.async_copy(tab_hbm.at[idx_v.at[0, j]], buf.at[b], sem[b],
                         add=add)

    def wait_gather(j, idx_v, sem):
        b = j % NBUF
        pltpu.make_async_copy(tab_hbm.at[idx_v.at[0, j]], buf.at[b],
                              sem[b]).wait()

    def scatter(j):
        b = j % NBUF
        pltpu.async_copy(buf.at[b], out_hbm.at[pl.ds((gbase + j) * CH, CH)],
                         gc[b])

    def wait_scatter(j):
        b = j % NBUF
        pltpu.make_async_copy(buf.at[b], out_hbm.at[pl.ds((gbase + j) * CH, CH)],
                              gc[b]).wait()

    for j in range(LOOK):
        gather(j, idx0_v, ga, False)
    for j in range(G):
        if j + LOOK < G:
            if j >= NBUF - LOOK:
                wait_scatter(j - (NBUF - LOOK))
            gather(j + LOOK, idx0_v, ga, False)
        wait_gather(j, idx0_v, ga)
        gather(j, idx1_v, gb, True)
        wait_gather(j, idx1_v, gb)
        scatter(j)
    for j in range(G - NBUF, G):
        wait_scatter(j)


def kernel(X, w_embed):
    Xf = X.reshape(N, 2).astype(jnp.int32)
    idx0 = Xf[:, 0].reshape(NW, G, CH)
    idx1 = Xf[:, 1].reshape(NW, G, CH)
    h = _sc_embed(idx0, idx1, w_embed)
    return h.reshape(B, T, D), w_embed
